# speedup vs baseline: 1.0001x; 1.0001x over previous
import math

import jax
import jax.numpy as jnp
from jax.experimental import pallas as pl
from jax.experimental.pallas import tpu as pltpu

N = 10000
DEG = 16
E = N * DEG
R = 2
B = 2
C_IN = 3
T_IN = 12
D_IN = C_IN * T_IN
D = 64
H = 4
DK = D // H
L = 2
T_OUT = 12


def _head_mm_body(h_ref, w_ref, b_ref, o_ref):
    o_ref[...] = (
        jnp.dot(h_ref[...], w_ref[...], preferred_element_type=jnp.float32)
        + b_ref[...]
    )


def _head_mm(h, w, b):
    return pl.pallas_call(
        _head_mm_body,
        out_shape=jax.ShapeDtypeStruct((N, T_OUT), jnp.float32),
    )(h, w, b.reshape(1, T_OUT))


def _edge_softmax(attn, d_idx):
    m = jax.ops.segment_max(attn, d_idx, num_segments=N)
    e = jnp.exp(attn - m[d_idx])
    s = jax.ops.segment_sum(e, d_idx, num_segments=N)
    return e / s[d_idx]


def _hgt_single(x, params, src, dst):
    h = jax.nn.gelu(x @ params['adapt_W'] + params['adapt_b'])
    for l in range(L):
        p = params['layers'][l]
        k = (h @ p['Wk'] + p['bk']).reshape(N, H, DK)
        q = (h @ p['Wq'] + p['bq']).reshape(N, H, DK)
        v = (h @ p['Wv'] + p['bv']).reshape(N, H, DK)
        rel_outs = []
        for r in range(R):
            kr = jnp.einsum('nhi,hij->nhj', k, p['w_att'][r])
            vr = jnp.einsum('nhi,hij->nhj', v, p['w_msg'][r])
            s_idx = src[r]
            d_idx = dst[r]
            t = jnp.sum(q[d_idx] * kr[s_idx], axis=-1)
            attn = t * p['mu'][r] / math.sqrt(DK)
            attn = _edge_softmax(attn, d_idx)
            msg = vr[s_idx] * attn[:, :, None]
            agg = jax.ops.segment_sum(msg, d_idx, num_segments=N)
            rel_outs.append(agg.reshape(N, D))
        hs = jnp.mean(jnp.stack(rel_outs, 0), axis=0)
        alpha = jax.nn.sigmoid(p['skip'])
        trans = hs @ p['Wa'] + p['ba']
        out = alpha * trans + (1.0 - alpha) * h
        mu_ = jnp.mean(out, axis=-1, keepdims=True)
        var = jnp.var(out, axis=-1, keepdims=True)
        h = (out - mu_) / jnp.sqrt(var + 1e-5) * p['ln_g'] + p['ln_b']
    return h @ params['pred_W'] + params['pred_b']


def kernel(data, timestamps, params, src, dst):
    x = data.reshape(B, N, D_IN)
    h = jax.vmap(lambda xb: _hgt_single(xb, params, src, dst))(x)
    outs = [
        _head_mm(h[b], params['head_W'], params['head_b']) for b in range(B)
    ]
    return jnp.stack(outs, 0)


# speedup vs baseline: 34.1073x; 34.1036x over previous
import math

import jax
import jax.numpy as jnp
from jax import lax
from jax.experimental import pallas as pl
from jax.experimental.pallas import tpu as pltpu
from jax.experimental.pallas import tpu_sc as plsc

N = 10000
DEG = 16
E = N * DEG
R = 2
B = 2
C_IN = 3
T_IN = 12
D_IN = C_IN * T_IN
D = 64
H = 4
DK = D // H
L = 2
T_OUT = 12

NCORE = 2
NSUB = 16
NT = NCORE * NSUB
NB = 320
NPAD = NT * NB
FLUSH = 2048
ECAP = E + FLUSH
CHB = 4000
CH = 128

_MESH = plsc.VectorSubcoreMesh(core_axis_name="c", subcore_axis_name="s")


def _bin_body(src_hbm, dst_hbm, lp0, lp1, cnts,
              stage_s, stage_d, buf, cnt_v):
    wid = lax.axis_index("c") * NSUB + lax.axis_index("s")
    base = wid * NB
    iota = lax.iota(jnp.int32, 16)
    zero16 = jnp.zeros((16,), jnp.int32)

    def zinit(i, _):
        buf[pl.ds(i * 16, 16)] = zero16
        return 0

    lax.fori_loop(0, (FLUSH + 16) // 16, zinit, 0)

    for r, lpr in enumerate((lp0, lp1)):
        def outer(cb, carry, lpr=lpr, r=r):
            cnt, written = carry
            off = pl.multiple_of(r * E + cb * CHB, 8)
            pltpu.sync_copy(src_hbm.at[pl.ds(off, CHB)], stage_s)
            pltpu.sync_copy(dst_hbm.at[pl.ds(off, CHB)], stage_d)

            def inner(i, carry2):
                cnt, written = carry2
                d16 = stage_d[pl.ds(i * 16, 16)]
                s16 = stage_s[pl.ds(i * 16, 16)]
                dl = d16 - base
                msk = (dl >= 0) & (dl < NB)
                dlc = jnp.clip(dl, 0, NB - 1)
                packed = s16 * 512 + dlc
                key = jnp.where(msk, iota, jnp.full((16,), 16, jnp.int32))
                _, sv = lax.sort((key, packed), num_keys=1)
                buf[pl.ds(cnt, 16)] = sv
                cnt = cnt + plsc.all_reduce_population_count(msk)[0]

                def do_flush(c, w):
                    wo = pl.multiple_of(wid * ECAP + w, 8)
                    pltpu.sync_copy(buf.at[pl.ds(0, FLUSH)],
                                    lpr.at[pl.ds(wo, FLUSH)])
                    rem = buf[pl.ds(FLUSH, 16)]
                    buf[pl.ds(0, 16)] = rem
                    return c - FLUSH, w + FLUSH

                cnt, written = lax.cond(cnt >= FLUSH, do_flush,
                                        lambda c, w: (c, w), cnt, written)
                return (cnt, written)

            return lax.fori_loop(0, CHB // 16, inner, (cnt, written))

        cnt, written = lax.fori_loop(0, E // CHB, outer,
                                     (jnp.int32(0), jnp.int32(0)))
        wo = pl.multiple_of(wid * ECAP + written, 8)
        pltpu.sync_copy(buf.at[pl.ds(0, FLUSH)], lpr.at[pl.ds(wo, FLUSH)])
        cnt_v[...] = jnp.broadcast_to(written + cnt, (16,)).astype(jnp.int32)
        co = pl.multiple_of((r * NT + wid) * 16, 8)
        pltpu.sync_copy(cnt_v, cnts.at[pl.ds(co, 16)])


def _bin_edges(src, dst):
    f = pl.kernel(
        _bin_body,
        out_type=(
            jax.ShapeDtypeStruct((NT * ECAP,), jnp.int32),
            jax.ShapeDtypeStruct((NT * ECAP,), jnp.int32),
            jax.ShapeDtypeStruct((R * NT * 16,), jnp.int32),
        ),
        mesh=_MESH,
        compiler_params=pltpu.CompilerParams(needs_layout_passes=False),
        scratch_types=[
            pltpu.VMEM((CHB,), jnp.int32),
            pltpu.VMEM((CHB,), jnp.int32),
            pltpu.VMEM((FLUSH + 16,), jnp.int32),
            pltpu.VMEM((16,), jnp.int32),
        ],
    )
    return f(src.reshape(R * E), dst.reshape(R * E))


def _edge_body(q_hbm, kr0, kr1, vr0, vr1, lp0, lp1, cnts,
               agg0, agg1, lg,
               q_loc, m_buf, s_buf, acc, pk_v, src_v, dl_v, kr_rows, vr_rows,
               tl_v, cnt_v, sem):
    wid = lax.axis_index("c") * NSUB + lax.axis_index("s")
    base = wid * NB
    iota = lax.iota(jnp.int32, 16)
    iota_h = jnp.minimum(iota, 3)
    zf = jnp.zeros((16,), jnp.float32)
    minf = jnp.full((16,), -3.0e38, jnp.float32)

    pltpu.sync_copy(q_hbm.at[pl.ds(pl.multiple_of(base * 128, 8), NB * 128)],
                    q_loc)

    for r in range(R):
        krr = (kr0, kr1)[r]
        vrr = (vr0, vr1)[r]
        lpr = (lp0, lp1)[r]
        aggr = (agg0, agg1)[r]

        co = pl.multiple_of((r * NT + wid) * 16, 8)
        pltpu.sync_copy(cnts.at[pl.ds(co, 16)], cnt_v)
        cnt = cnt_v[...][0]

        def init_ms(i, _):
            m_buf[pl.ds(i * 16, 16)] = minf
            s_buf[pl.ds(i * 16, 16)] = zf
            return 0

        lax.fori_loop(0, NB, init_ms, 0)

        def init_acc(i, _):
            acc[pl.ds(i * 16, 16)] = zf
            return 0

        lax.fori_loop(0, NB * 4, init_acc, 0)

        nchunks = (cnt + CH - 1) // CH

        def p1(ci, _, lpr=lpr, krr=krr, cnt=cnt):
            off = ci * CH
            lo = pl.multiple_of(wid * ECAP + off, 8)
            pltpu.sync_copy(lpr.at[pl.ds(lo, CH)], pk_v)
            def unpk(u, _):
                pv = pk_v[pl.ds(u * 16, 16)]
                src_v[pl.ds(u * 16, 16)] = jnp.right_shift(pv, 9)
                dl_v[pl.ds(u * 16, 16)] = jnp.bitwise_and(pv, 511)
                return 0

            lax.fori_loop(0, CH // 16, unpk, 0)
            pltpu.async_copy(krr.at[src_v], kr_rows, sem).wait()

            def sub(sc, _):
                e0 = sc * 16
                eidx = iota + e0
                dl16 = dl_v[pl.ds(e0, 16)]
                dlb = dl16 * 128
                t = [zf, zf, zf, zf]
                for c in range(64):
                    csp = jnp.full((16,), c, jnp.int32)
                    qv = plsc.load_gather(q_loc, [dlb + csp])
                    kv = plsc.load_gather(kr_rows, [eidx, csp])
                    t[c // 16] = t[c // 16] + qv * kv
                for hh in range(4):
                    tl_v[pl.ds(hh * CH + e0, 16)] = t[hh]
                for j in range(16):
                    valid = (off + e0 + j) < cnt
                    dlj = dl16[j]
                    esp = jnp.broadcast_to(e0 + j, (16,)).astype(jnp.int32)
                    tv = plsc.load_gather(tl_v, [iota_h * CH + esp])
                    tv = jnp.where(valid, tv, minf)
                    mrow = m_buf[pl.ds(dlj * 16, 16)]
                    m_buf[pl.ds(dlj * 16, 16)] = jnp.maximum(mrow, tv)
                return 0

            lax.fori_loop(0, CH // 16, sub, 0)
            go = pl.multiple_of((wid * E + off) * 4, 8)
            pltpu.sync_copy(tl_v, lg.at[pl.ds(go, 4 * CH)])
            return 0

        lax.fori_loop(0, nchunks, p1, 0)

        def p2(ci, _, lpr=lpr, vrr=vrr, cnt=cnt):
            off = ci * CH
            lo = pl.multiple_of(wid * ECAP + off, 8)
            pltpu.sync_copy(lpr.at[pl.ds(lo, CH)], pk_v)
            def unpk(u, _):
                pv = pk_v[pl.ds(u * 16, 16)]
                src_v[pl.ds(u * 16, 16)] = jnp.right_shift(pv, 9)
                dl_v[pl.ds(u * 16, 16)] = jnp.bitwise_and(pv, 511)
                return 0

            lax.fori_loop(0, CH // 16, unpk, 0)
            go = pl.multiple_of((wid * E + off) * 4, 8)
            pltpu.sync_copy(lg.at[pl.ds(go, 4 * CH)], tl_v)
            pltpu.async_copy(vrr.at[src_v], vr_rows, sem).wait()

            def sub2(sc, _):
                e0 = sc * 16
                dl16 = dl_v[pl.ds(e0, 16)]
                dlm = dl16 * 16
                for hh in range(4):
                    tv = tl_v[pl.ds(hh * CH + e0, 16)]
                    hsp = jnp.full((16,), hh, jnp.int32)
                    mv = plsc.load_gather(m_buf, [dlm + hsp])
                    tl_v[pl.ds(hh * CH + e0, 16)] = jnp.exp(tv - mv)
                for j in range(16):
                    valid = (off + e0 + j) < cnt
                    dlj = dl16[j]
                    esp = jnp.broadcast_to(e0 + j, (16,)).astype(jnp.int32)
                    evec = plsc.load_gather(tl_v, [iota_h * CH + esp])
                    evec = jnp.where(valid, evec, zf)
                    plsc.addupdate(s_buf.at[pl.ds(dlj * 16, 16)], evec)
                    for hh in range(4):
                        e_h = evec[hh]
                        vrv = vr_rows[e0 + j, pl.ds(hh * 16, 16)]
                        plsc.addupdate(
                            acc.at[pl.ds(dlj * 64 + hh * 16, 16)],
                            vrv * e_h)
                return 0

            lax.fori_loop(0, CH // 16, sub2, 0)
            return 0

        lax.fori_loop(0, nchunks, p2, 0)

        def fin(i, _):
            srow = s_buf[pl.ds(i * 16, 16)]
            for hh in range(4):
                sv = srow[hh]
                sv = jnp.where(sv > 0.0, sv, 1.0)
                o = pl.ds(i * 64 + hh * 16, 16)
                acc[o] = acc[o] / sv
            return 0

        lax.fori_loop(0, NB, fin, 0)
        pltpu.sync_copy(
            acc, aggr.at[pl.ds(pl.multiple_of(base * 64, 8), NB * 64)])


def _edge_pass(qp, kr0, kr1, vr0, vr1, lp0, lp1, cnts):
    f = pl.kernel(
        _edge_body,
        out_type=(
            jax.ShapeDtypeStruct((NPAD * D,), jnp.float32),
            jax.ShapeDtypeStruct((NPAD * D,), jnp.float32),
            jax.ShapeDtypeStruct((NT * H * E,), jnp.float32),
        ),
        mesh=_MESH,
        compiler_params=pltpu.CompilerParams(needs_layout_passes=False),
        scratch_types=[
            pltpu.VMEM((NB * 2 * D,), jnp.float32),
            pltpu.VMEM((NB * 16,), jnp.float32),
            pltpu.VMEM((NB * 16,), jnp.float32),
            pltpu.VMEM((NB * D,), jnp.float32),
            pltpu.VMEM((CH,), jnp.int32),
            pltpu.VMEM((CH,), jnp.int32),
            pltpu.VMEM((CH,), jnp.int32),
            pltpu.VMEM((CH, 2 * D), jnp.float32),
            pltpu.VMEM((CH, 2 * D), jnp.float32),
            pltpu.VMEM((H * CH,), jnp.float32),
            pltpu.VMEM((16,), jnp.int32),
            pltpu.SemaphoreType.DMA,
        ],
    )
    a0, a1, _ = f(qp, kr0, kr1, vr0, vr1, lp0, lp1, cnts)
    return a0.reshape(NPAD, D), a1.reshape(NPAD, D)


def _pre_body(x_ref, w_ref, b_ref, o_ref):
    h = jnp.dot(x_ref[...], w_ref[...],
                preferred_element_type=jnp.float32) + b_ref[...]
    o_ref[...] = jax.nn.gelu(h)


def _pre(xp, w, b):
    return pl.pallas_call(
        _pre_body,
        out_shape=jax.ShapeDtypeStruct((NPAD, D), jnp.float32),
    )(xp, w, b.reshape(1, D))


def _qkv_body(h_ref, w_ref, b_ref, o_ref):
    res = jnp.dot(h_ref[...], w_ref[0],
                  preferred_element_type=jnp.float32) + b_ref[0]
    o_ref[0] = jnp.concatenate([res, jnp.zeros_like(res)], axis=-1)


def _qkv(hp, wstack, bstack):
    nmat = wstack.shape[0]
    bstack = bstack.reshape(nmat, 1, D)
    return pl.pallas_call(
        _qkv_body,
        grid=(nmat,),
        in_specs=[
            pl.BlockSpec((NPAD, D), lambda i: (0, 0)),
            pl.BlockSpec((1, D, D), lambda i: (i, 0, 0)),
            pl.BlockSpec((1, 1, D), lambda i: (i, 0, 0)),
        ],
        out_specs=pl.BlockSpec((1, NPAD, 2 * D), lambda i: (i, 0, 0)),
        out_shape=jax.ShapeDtypeStruct((nmat, NPAD, 2 * D), jnp.float32),
    )(hp, wstack, bstack)


def _combine_body(a0_ref, a1_ref, h_ref, wa_ref, ba_ref, g_ref, bb_ref,
                  skip_ref, o_ref):
    hs = (a0_ref[...] + a1_ref[...]) * 0.5
    trans = jnp.dot(hs, wa_ref[...],
                    preferred_element_type=jnp.float32) + ba_ref[...]
    alpha = jax.nn.sigmoid(skip_ref[0, 0])
    out = alpha * trans + (1.0 - alpha) * h_ref[...]
    mu = jnp.mean(out, axis=-1, keepdims=True)
    d = out - mu
    var = jnp.mean(d * d, axis=-1, keepdims=True)
    o_ref[...] = d * lax.rsqrt(var + 1e-5) * g_ref[...] + bb_ref[...]


def _combine(a0, a1, hp, wa, ba, g, bb, skip):
    return pl.pallas_call(
        _combine_body,
        out_shape=jax.ShapeDtypeStruct((NPAD, D), jnp.float32),
    )(a0, a1, hp, wa, ba.reshape(1, D), g.reshape(1, D), bb.reshape(1, D),
      skip.reshape(1, 1))


def _final_body(h_ref, wp_ref, bp_ref, wh_ref, bh_ref, o_ref):
    t = jnp.dot(h_ref[...], wp_ref[...],
                preferred_element_type=jnp.float32) + bp_ref[...]
    o_ref[...] = jnp.dot(t, wh_ref[...],
                         preferred_element_type=jnp.float32) + bh_ref[...]


def _final(hp, wp, bp, wh, bh):
    return pl.pallas_call(
        _final_body,
        out_shape=jax.ShapeDtypeStruct((NPAD, T_OUT), jnp.float32),
    )(hp, wp, bp.reshape(1, D), wh, bh.reshape(1, T_OUT))


def _block_diag(w):
    out = jnp.zeros((D, D), jnp.float32)
    for hh in range(H):
        out = out.at[hh * DK:(hh + 1) * DK, hh * DK:(hh + 1) * DK].set(w[hh])
    return out


def kernel(data, timestamps, params, src, dst):
    x = data.reshape(B, N, D_IN)
    xp = jnp.pad(x, ((0, 0), (0, NPAD - N), (0, 0)))

    lp0, lp1, cnts = _bin_edges(src, dst)

    layer_w = []
    for l in range(L):
        p = params['layers'][l]
        ws, bs = [p['Wq']], [p['bq']]
        for r in range(R):
            bd = _block_diag(p['w_att'][r])
            scale = jnp.repeat(p['mu'][r], DK) / math.sqrt(DK)
            ws.append((p['Wk'] @ bd) * scale[None, :])
            bs.append((p['bk'] @ bd) * scale)
        for r in range(R):
            bd = _block_diag(p['w_msg'][r])
            ws.append(p['Wv'] @ bd)
            bs.append(p['bv'] @ bd)
        layer_w.append((jnp.stack(ws), jnp.stack(bs)))

    outs = []
    for b in range(B):
        hp = _pre(xp[b], params['adapt_W'], params['adapt_b'])
        for l in range(L):
            p = params['layers'][l]
            wstack, bstack = layer_w[l]
            proj = _qkv(hp, wstack, bstack)
            a0, a1 = _edge_pass(proj[0].reshape(NPAD * 2 * D), proj[1],
                                proj[2], proj[3], proj[4], lp0, lp1, cnts)
            hp = _combine(a0, a1, hp, p['Wa'], p['ba'], p['ln_g'], p['ln_b'],
                          p['skip'])
        outs.append(_final(hp, params['pred_W'], params['pred_b'],
                           params['head_W'], params['head_b'])[:N])
    return jnp.stack(outs, 0)
